# baseline (speedup 1.0000x reference)
import jax
import jax.numpy as jnp
from jax.experimental import pallas as pl
from jax.experimental.pallas import tpu as pltpu

_R = 4096
_C = 256


def _scan_tile(x_ref, s_ref, out_ref, carry_ref):
    j = pl.program_id(1)

    @pl.when(j == 0)
    def _init():
        carry_ref[...] = jnp.zeros_like(carry_ref)

    x = x_ref[...]
    partial = jax.lax.dot_general(
        x, s_ref[...],
        dimension_numbers=(((1,), (0,)), ((), ())),
        preferred_element_type=jnp.float32,
        precision=jax.lax.Precision.DEFAULT,
    )
    out_ref[...] = partial + carry_ref[...]
    carry_ref[...] += jnp.sum(x, axis=1, keepdims=True)


def kernel(x):
    n_rows, n_cols = x.shape
    grid = (n_rows // _R, n_cols // _C)
    ii = jax.lax.broadcasted_iota(jnp.int32, (_C, _C), 0)
    kk = jax.lax.broadcasted_iota(jnp.int32, (_C, _C), 1)
    s = (ii < kk).astype(jnp.float32)
    return pl.pallas_call(
        _scan_tile,
        grid=grid,
        in_specs=[
            pl.BlockSpec((_R, _C), lambda i, j: (i, j)),
            pl.BlockSpec((_C, _C), lambda i, j: (0, 0)),
        ],
        out_specs=pl.BlockSpec((_R, _C), lambda i, j: (i, j)),
        out_shape=jax.ShapeDtypeStruct(x.shape, x.dtype),
        scratch_shapes=[pltpu.VMEM((_R, 1), jnp.float32)],
        compiler_params=pltpu.CompilerParams(
            dimension_semantics=("parallel", "arbitrary"),
        ),
    )(x, s)


# speedup vs baseline: 1.0041x; 1.0041x over previous
import jax
import jax.numpy as jnp
from jax.experimental import pallas as pl
from jax.experimental.pallas import tpu as pltpu

_R = 4096
_C = 512


def _scan_tile(x_ref, s_ref, out_ref, carry_ref):
    j = pl.program_id(1)

    @pl.when(j == 0)
    def _init():
        carry_ref[...] = jnp.zeros_like(carry_ref)

    x = x_ref[...]
    partial = jax.lax.dot_general(
        x, s_ref[...],
        dimension_numbers=(((1,), (0,)), ((), ())),
        preferred_element_type=jnp.float32,
        precision=jax.lax.Precision.DEFAULT,
    )
    out_ref[...] = partial + carry_ref[...]
    carry_ref[...] += jnp.sum(x, axis=1, keepdims=True)


def kernel(x):
    n_rows, n_cols = x.shape
    grid = (n_rows // _R, n_cols // _C)
    ii = jax.lax.broadcasted_iota(jnp.int32, (_C, _C), 0)
    kk = jax.lax.broadcasted_iota(jnp.int32, (_C, _C), 1)
    s = (ii < kk).astype(jnp.float32)
    return pl.pallas_call(
        _scan_tile,
        grid=grid,
        in_specs=[
            pl.BlockSpec((_R, _C), lambda i, j: (i, j)),
            pl.BlockSpec((_C, _C), lambda i, j: (0, 0)),
        ],
        out_specs=pl.BlockSpec((_R, _C), lambda i, j: (i, j)),
        out_shape=jax.ShapeDtypeStruct(x.shape, x.dtype),
        scratch_shapes=[pltpu.VMEM((_R, 1), jnp.float32)],
        compiler_params=pltpu.CompilerParams(
            dimension_semantics=("parallel", "arbitrary"),
        ),
    )(x, s)
